# baseline (speedup 1.0000x reference)
import jax
import jax.numpy as jnp
from jax import lax
from jax.experimental import pallas as pl
from jax.experimental.pallas import tpu as pltpu
from jax.experimental.pallas import tpu_sc as plsc

EMBED_DIM = 32
BATCH = 16384
NUM_COLS = 1000000
FLAT = EMBED_DIM * NUM_COLS
NUM_CORES = 2
NUM_WORKERS = 32
ROWS_PER_WORKER = BATCH // NUM_WORKERS
LANES = 16
GROUPS = ROWS_PER_WORKER // LANES


def _body(uids_ref, iids_ref, ut_ref, it_ref,
          out_ref, uids_v, iids_v, idx_u, idx_i, ubuf, ibuf, out_v,
          sem_u, sem_i):
    wid = lax.axis_index("s") * NUM_CORES + lax.axis_index("c")
    base = wid * ROWS_PER_WORKER

    pltpu.sync_copy(uids_ref.at[pl.ds(base, ROWS_PER_WORKER)], uids_v)
    pltpu.sync_copy(iids_ref.at[pl.ds(base, ROWS_PER_WORKER)], iids_v)

    def group_body(g, _):
        cu = uids_v[pl.ds(g * LANES, LANES)]
        ci = iids_v[pl.ds(g * LANES, LANES)]
        for d in range(EMBED_DIM):
            sl = pl.ds(16 * d, LANES)
            idx_u[sl] = cu + d * NUM_COLS
            idx_i[sl] = ci + d * NUM_COLS
        for j in range(4):
            pltpu.async_copy(ut_ref.at[idx_u.at[pl.ds(128 * j, 128)]],
                             ubuf.at[pl.ds(128 * j, 128)], sem_u)
            pltpu.async_copy(it_ref.at[idx_i.at[pl.ds(128 * j, 128)]],
                             ibuf.at[pl.ds(128 * j, 128)], sem_i)
        for j in range(4):
            pltpu.make_async_copy(ut_ref.at[pl.ds(0, 128)],
                                  ubuf.at[pl.ds(128 * j, 128)], sem_u).wait()
            pltpu.make_async_copy(it_ref.at[pl.ds(0, 128)],
                                  ibuf.at[pl.ds(128 * j, 128)], sem_i).wait()

        acc = ubuf[pl.ds(0, LANES)] * ibuf[pl.ds(0, LANES)]
        for d in range(1, EMBED_DIM):
            sl = pl.ds(16 * d, LANES)
            acc = acc + ubuf[sl] * ibuf[sl]
        out_v[pl.ds(g * LANES, LANES)] = 5.0 / (1.0 + jnp.exp(-acc))
        return 0

    lax.fori_loop(0, GROUPS, group_body, 0)

    pltpu.sync_copy(out_v, out_ref.at[wid])


def kernel(user_ids, item_ids, user_table, item_table):
    uids = user_ids.astype(jnp.int32)
    iids = item_ids.astype(jnp.int32)
    ut = user_table.T.reshape(FLAT)
    it = item_table.T.reshape(FLAT)

    mesh = plsc.VectorSubcoreMesh(core_axis_name="c", subcore_axis_name="s")
    f = pl.kernel(
        _body,
        out_type=jax.ShapeDtypeStruct((NUM_WORKERS, ROWS_PER_WORKER),
                                      jnp.float32),
        mesh=mesh,
        scratch_types=[
            pltpu.VMEM((ROWS_PER_WORKER,), jnp.int32),
            pltpu.VMEM((ROWS_PER_WORKER,), jnp.int32),
            pltpu.VMEM((512,), jnp.int32),
            pltpu.VMEM((512,), jnp.int32),
            pltpu.VMEM((512,), jnp.float32),
            pltpu.VMEM((512,), jnp.float32),
            pltpu.VMEM((ROWS_PER_WORKER,), jnp.float32),
            pltpu.SemaphoreType.DMA,
            pltpu.SemaphoreType.DMA,
        ],
        compiler_params=pltpu.CompilerParams(use_tc_tiling_on_sc=False),
    )
    return f(uids, iids, ut, it).reshape(BATCH)


# speedup vs baseline: 15.2116x; 15.2116x over previous
import jax
import jax.numpy as jnp
from jax import lax
from jax.experimental import pallas as pl
from jax.experimental.pallas import tpu as pltpu
from jax.experimental.pallas import tpu_sc as plsc

EMBED_DIM = 32
BATCH = 16384
NUM_CORES = 2
NUM_WORKERS = 32
ROWS_PER_WORKER = BATCH // NUM_WORKERS
LANES = 16
GROUPS = ROWS_PER_WORKER // LANES
CHUNK = 8
CHUNKS = ROWS_PER_WORKER // CHUNK


def _body(uids_ref, iids_ref, ut_ref, it_ref,
          out_ref, uids_v, iids_v, utile, itile, out_v, sem_u, sem_i):
    wid = lax.axis_index("s") * NUM_CORES + lax.axis_index("c")
    base = wid * ROWS_PER_WORKER

    pltpu.sync_copy(uids_ref.at[pl.ds(base, ROWS_PER_WORKER)], uids_v)
    pltpu.sync_copy(iids_ref.at[pl.ds(base, ROWS_PER_WORKER)], iids_v)

    lane = lax.iota(jnp.int32, LANES)
    perms = [lane ^ (1 << b) for b in range(4)]

    def shuffle(x, perm):
        return lax.gather(
            x, perm[:, None],
            lax.GatherDimensionNumbers(
                offset_dims=(), collapsed_slice_dims=(0,),
                start_index_map=(0,)),
            slice_sizes=(1,),
            mode=lax.GatherScatterMode.PROMISE_IN_BOUNDS)

    def group_body(g, _):
        cu_vec = uids_v[pl.ds(g * LANES, LANES)]
        ci_vec = iids_v[pl.ds(g * LANES, LANES)]
        acc = jnp.zeros((LANES,), jnp.float32)

        for k0 in (0, CHUNK):
            for k in range(CHUNK):
                cu = cu_vec[k0 + k]
                ci = ci_vec[k0 + k]
                off_u = pl.multiple_of((cu >> 7) * 128, 128)
                off_i = pl.multiple_of((ci >> 7) * 128, 128)
                pltpu.async_copy(ut_ref.at[:, pl.ds(off_u, 128)],
                                 utile.at[k], sem_u)
                pltpu.async_copy(it_ref.at[:, pl.ds(off_i, 128)],
                                 itile.at[k], sem_i)
            for k in range(CHUNK):
                pltpu.make_async_copy(ut_ref.at[:, pl.ds(0, 128)],
                                      utile.at[k], sem_u).wait()
                pltpu.make_async_copy(it_ref.at[:, pl.ds(0, 128)],
                                      itile.at[k], sem_i).wait()

            for k in range(CHUNK):
                cu = cu_vec[k0 + k]
                ci = ci_vec[k0 + k]
                pu = jnp.full((LANES,), cu & 15, jnp.int32)
                pi = jnp.full((LANES,), ci & 15, jnp.int32)
                o16u = ((cu >> 4) & 7) * 16
                o16i = ((ci >> 4) & 7) * 16
                s = jnp.zeros((LANES,), jnp.float32)
                for d in range(EMBED_DIM):
                    uv = utile[k, d, pl.ds(o16u, LANES)]
                    iv = itile[k, d, pl.ds(o16i, LANES)]
                    s = s + shuffle(uv, pu) * iv
                acc = jnp.where(lane == k0 + k, shuffle(s, pi), acc)

        out_v[pl.ds(g * LANES, LANES)] = 5.0 / (1.0 + jnp.exp(-acc))
        return 0

    lax.fori_loop(0, GROUPS, group_body, 0)

    pltpu.sync_copy(out_v, out_ref.at[wid])


def kernel(user_ids, item_ids, user_table, item_table):
    uids = user_ids.astype(jnp.int32)
    iids = item_ids.astype(jnp.int32)
    ut = user_table.T
    it = item_table.T

    mesh = plsc.VectorSubcoreMesh(core_axis_name="c", subcore_axis_name="s")
    f = pl.kernel(
        _body,
        out_type=jax.ShapeDtypeStruct((NUM_WORKERS, ROWS_PER_WORKER),
                                      jnp.float32),
        mesh=mesh,
        scratch_types=[
            pltpu.VMEM((ROWS_PER_WORKER,), jnp.int32),
            pltpu.VMEM((ROWS_PER_WORKER,), jnp.int32),
            pltpu.VMEM((CHUNK, EMBED_DIM, 128), jnp.float32),
            pltpu.VMEM((CHUNK, EMBED_DIM, 128), jnp.float32),
            pltpu.VMEM((ROWS_PER_WORKER,), jnp.float32),
            pltpu.SemaphoreType.DMA,
            pltpu.SemaphoreType.DMA,
        ],
        compiler_params=pltpu.CompilerParams(use_tc_tiling_on_sc=True),
    )
    return f(uids, iids, ut, it).reshape(BATCH)


# speedup vs baseline: 18.1316x; 1.1920x over previous
import jax
import jax.numpy as jnp
from jax import lax
from jax.experimental import pallas as pl
from jax.experimental.pallas import tpu as pltpu
from jax.experimental.pallas import tpu_sc as plsc

EMBED_DIM = 32
BATCH = 16384
NUM_CORES = 2
NUM_WORKERS = 32
ROWS_PER_WORKER = BATCH // NUM_WORKERS
LANES = 16
GROUPS = ROWS_PER_WORKER // LANES
CHUNK = 4
CHUNKS = ROWS_PER_WORKER // CHUNK


def _body(uids_ref, iids_ref, ut_ref, it_ref,
          out_ref, uids_v, iids_v, utile, itile, out_v, sem_u, sem_i):
    wid = lax.axis_index("s") * NUM_CORES + lax.axis_index("c")
    base = wid * ROWS_PER_WORKER

    pltpu.sync_copy(uids_ref.at[pl.ds(base, ROWS_PER_WORKER)], uids_v)
    pltpu.sync_copy(iids_ref.at[pl.ds(base, ROWS_PER_WORKER)], iids_v)

    lane = lax.iota(jnp.int32, LANES)
    perms = [lane ^ (1 << b) for b in range(4)]

    def shuffle(x, perm):
        return lax.gather(
            x, perm[:, None],
            lax.GatherDimensionNumbers(
                offset_dims=(), collapsed_slice_dims=(0,),
                start_index_map=(0,)),
            slice_sizes=(1,),
            mode=lax.GatherScatterMode.PROMISE_IN_BOUNDS)

    def group_body(g, _):
        cu_vec = uids_v[pl.ds(g * LANES, LANES)]
        ci_vec = iids_v[pl.ds(g * LANES, LANES)]

        def fire_chunk(p, k0):
            for k in range(CHUNK):
                cu = cu_vec[k0 + k]
                ci = ci_vec[k0 + k]
                off_u = pl.multiple_of((cu >> 7) * 128, 128)
                off_i = pl.multiple_of((ci >> 7) * 128, 128)
                pltpu.async_copy(ut_ref.at[:, pl.ds(off_u, 128)],
                                 utile.at[p, k], sem_u)
                pltpu.async_copy(it_ref.at[:, pl.ds(off_i, 128)],
                                 itile.at[p, k], sem_i)

        def wait_chunk(p):
            for k in range(CHUNK):
                pltpu.make_async_copy(ut_ref.at[:, pl.ds(0, 128)],
                                      utile.at[p, k], sem_u).wait()
                pltpu.make_async_copy(it_ref.at[:, pl.ds(0, 128)],
                                      itile.at[p, k], sem_i).wait()

        def extract_chunk(p, k0, acc):
            for k in range(CHUNK):
                cu = cu_vec[k0 + k]
                ci = ci_vec[k0 + k]
                pu = jnp.full((LANES,), cu & 15, jnp.int32)
                pi = jnp.full((LANES,), ci & 15, jnp.int32)
                o16u = ((cu >> 4) & 7) * 16
                o16i = ((ci >> 4) & 7) * 16
                s = jnp.zeros((LANES,), jnp.float32)
                for d in range(EMBED_DIM):
                    uv = utile[p, k, d, pl.ds(o16u, LANES)]
                    iv = itile[p, k, d, pl.ds(o16i, LANES)]
                    s = s + shuffle(uv, pu) * iv
                acc = jnp.where(lane == k0 + k, shuffle(s, pi), acc)
            return acc

        acc = jnp.zeros((LANES,), jnp.float32)
        n_chunks = LANES // CHUNK
        fire_chunk(0, 0)
        for c in range(n_chunks):
            if c + 1 < n_chunks:
                fire_chunk((c + 1) % 2, (c + 1) * CHUNK)
            wait_chunk(c % 2)
            acc = extract_chunk(c % 2, c * CHUNK, acc)

        out_v[pl.ds(g * LANES, LANES)] = 5.0 / (1.0 + jnp.exp(-acc))
        return 0

    lax.fori_loop(0, GROUPS, group_body, 0)

    pltpu.sync_copy(out_v, out_ref.at[wid])


def kernel(user_ids, item_ids, user_table, item_table):
    uids = user_ids.astype(jnp.int32)
    iids = item_ids.astype(jnp.int32)
    ut = user_table.T
    it = item_table.T

    mesh = plsc.VectorSubcoreMesh(core_axis_name="c", subcore_axis_name="s")
    f = pl.kernel(
        _body,
        out_type=jax.ShapeDtypeStruct((NUM_WORKERS, ROWS_PER_WORKER),
                                      jnp.float32),
        mesh=mesh,
        scratch_types=[
            pltpu.VMEM((ROWS_PER_WORKER,), jnp.int32),
            pltpu.VMEM((ROWS_PER_WORKER,), jnp.int32),
            pltpu.VMEM((2, CHUNK, EMBED_DIM, 128), jnp.float32),
            pltpu.VMEM((2, CHUNK, EMBED_DIM, 128), jnp.float32),
            pltpu.VMEM((ROWS_PER_WORKER,), jnp.float32),
            pltpu.SemaphoreType.DMA,
            pltpu.SemaphoreType.DMA,
        ],
        compiler_params=pltpu.CompilerParams(use_tc_tiling_on_sc=True),
    )
    return f(uids, iids, ut, it).reshape(BATCH)


# speedup vs baseline: 18.9863x; 1.0471x over previous
import jax
import jax.numpy as jnp
from jax import lax
from jax.experimental import pallas as pl
from jax.experimental.pallas import tpu as pltpu
from jax.experimental.pallas import tpu_sc as plsc

EMBED_DIM = 32
BATCH = 16384
NUM_CORES = 2
NUM_WORKERS = 32
ROWS_PER_WORKER = BATCH // NUM_WORKERS
LANES = 16
GROUPS = ROWS_PER_WORKER // LANES
CHUNK = 4
CHUNKS = ROWS_PER_WORKER // CHUNK


def _body(uids_ref, iids_ref, ut_ref, it_ref,
          out_ref, uids_v, iids_v, utile, itile, out_v, sem_u, sem_i):
    wid = lax.axis_index("s") * NUM_CORES + lax.axis_index("c")
    base = wid * ROWS_PER_WORKER

    pltpu.sync_copy(uids_ref.at[pl.ds(base, ROWS_PER_WORKER)], uids_v)
    pltpu.sync_copy(iids_ref.at[pl.ds(base, ROWS_PER_WORKER)], iids_v)

    lane = lax.iota(jnp.int32, LANES)
    perms = [lane ^ (1 << b) for b in range(4)]

    def shuffle(x, perm):
        return lax.gather(
            x, perm[:, None],
            lax.GatherDimensionNumbers(
                offset_dims=(), collapsed_slice_dims=(0,),
                start_index_map=(0,)),
            slice_sizes=(1,),
            mode=lax.GatherScatterMode.PROMISE_IN_BOUNDS)

    def group_body(g2, _):
        g = g2 * 2
        vec_u = [uids_v[pl.ds((g + i) * LANES, LANES)] for i in range(2)]
        vec_i = [iids_v[pl.ds((g + i) * LANES, LANES)] for i in range(2)]

        def ids_at(c, k):
            half = c // 4
            j = (c % 4) * CHUNK + k
            return vec_u[half][j], vec_i[half][j]

        def fire_chunk(c):
            p = c % 3
            for k in range(CHUNK):
                cu, ci = ids_at(c, k)
                off_u = pl.multiple_of((cu >> 7) * 128, 128)
                off_i = pl.multiple_of((ci >> 7) * 128, 128)
                pltpu.async_copy(ut_ref.at[:, pl.ds(off_u, 128)],
                                 utile.at[p, k], sem_u)
                pltpu.async_copy(it_ref.at[:, pl.ds(off_i, 128)],
                                 itile.at[p, k], sem_i)

        def wait_chunk(c):
            p = c % 3
            for k in range(CHUNK):
                pltpu.make_async_copy(ut_ref.at[:, pl.ds(0, 128)],
                                      utile.at[p, k], sem_u).wait()
                pltpu.make_async_copy(it_ref.at[:, pl.ds(0, 128)],
                                      itile.at[p, k], sem_i).wait()

        def extract_chunk(c, acc):
            p = c % 3
            for k in range(CHUNK):
                cu, ci = ids_at(c, k)
                pu = jnp.full((LANES,), cu & 15, jnp.int32)
                pi = jnp.full((LANES,), ci & 15, jnp.int32)
                o16u = ((cu >> 4) & 7) * 16
                o16i = ((ci >> 4) & 7) * 16
                s = jnp.zeros((LANES,), jnp.float32)
                for d in range(EMBED_DIM):
                    uv = utile[p, k, d, pl.ds(o16u, LANES)]
                    iv = itile[p, k, d, pl.ds(o16i, LANES)]
                    s = s + shuffle(uv, pu) * iv
                acc = jnp.where(lane == (c % 4) * CHUNK + k,
                                shuffle(s, pi), acc)
            return acc

        accs = [jnp.zeros((LANES,), jnp.float32) for _ in range(2)]
        fire_chunk(0)
        fire_chunk(1)
        for c in range(8):
            if c + 2 < 8:
                fire_chunk(c + 2)
            wait_chunk(c)
            accs[c // 4] = extract_chunk(c, accs[c // 4])

        for i in range(2):
            out_v[pl.ds((g + i) * LANES, LANES)] = (
                5.0 / (1.0 + jnp.exp(-accs[i])))
        return 0

    lax.fori_loop(0, GROUPS // 2, group_body, 0)

    pltpu.sync_copy(out_v, out_ref.at[wid])


def kernel(user_ids, item_ids, user_table, item_table):
    uids = user_ids.astype(jnp.int32)
    iids = item_ids.astype(jnp.int32)
    ut = user_table.T
    it = item_table.T

    mesh = plsc.VectorSubcoreMesh(core_axis_name="c", subcore_axis_name="s")
    f = pl.kernel(
        _body,
        out_type=jax.ShapeDtypeStruct((NUM_WORKERS, ROWS_PER_WORKER),
                                      jnp.float32),
        mesh=mesh,
        scratch_types=[
            pltpu.VMEM((ROWS_PER_WORKER,), jnp.int32),
            pltpu.VMEM((ROWS_PER_WORKER,), jnp.int32),
            pltpu.VMEM((3, CHUNK, EMBED_DIM, 128), jnp.float32),
            pltpu.VMEM((3, CHUNK, EMBED_DIM, 128), jnp.float32),
            pltpu.VMEM((ROWS_PER_WORKER,), jnp.float32),
            pltpu.SemaphoreType.DMA,
            pltpu.SemaphoreType.DMA,
        ],
        compiler_params=pltpu.CompilerParams(use_tc_tiling_on_sc=True),
    )
    return f(uids, iids, ut, it).reshape(BATCH)


# speedup vs baseline: 19.0412x; 1.0029x over previous
import jax
import jax.numpy as jnp
from jax import lax
from jax.experimental import pallas as pl
from jax.experimental.pallas import tpu as pltpu
from jax.experimental.pallas import tpu_sc as plsc

EMBED_DIM = 32
BATCH = 16384
NUM_CORES = 2
NUM_WORKERS = 32
ROWS_PER_WORKER = BATCH // NUM_WORKERS
LANES = 16
GROUPS = ROWS_PER_WORKER // LANES
CHUNK = 4


def _body(uids_ref, iids_ref, ut_ref, it_ref,
          out_ref, uids_v, iids_v, utile, itile, out_v, sem_u, sem_i):
    wid = lax.axis_index("s") * NUM_CORES + lax.axis_index("c")
    base = wid * ROWS_PER_WORKER

    pltpu.sync_copy(uids_ref.at[pl.ds(base, ROWS_PER_WORKER)], uids_v)
    pltpu.sync_copy(iids_ref.at[pl.ds(base, ROWS_PER_WORKER)], iids_v)

    lane = lax.iota(jnp.int32, LANES)

    def shuffle(x, perm):
        return lax.gather(
            x, perm[:, None],
            lax.GatherDimensionNumbers(
                offset_dims=(), collapsed_slice_dims=(0,),
                start_index_map=(0,)),
            slice_sizes=(1,),
            mode=lax.GatherScatterMode.PROMISE_IN_BOUNDS)

    def group_body(g2, _):
        g = g2 * 2
        vec_u = [uids_v[pl.ds((g + i) * LANES, LANES)] for i in range(2)]
        vec_i = [iids_v[pl.ds((g + i) * LANES, LANES)] for i in range(2)]

        def ids_at(c, k):
            half = c // 4
            j = (c % 4) * CHUNK + k
            return vec_u[half][j], vec_i[half][j]

        def fire_chunk(c):
            p = c % 3
            for k in range(CHUNK):
                cu, ci = ids_at(c, k)
                off_u = pl.multiple_of((cu >> 7) * 128, 128)
                off_i = pl.multiple_of((ci >> 7) * 128, 128)
                pltpu.async_copy(ut_ref.at[:, pl.ds(off_u, 128)],
                                 utile.at[p, k], sem_u)
                pltpu.async_copy(it_ref.at[:, pl.ds(off_i, 128)],
                                 itile.at[p, k], sem_i)

        def wait_chunk(c):
            p = c % 3
            for k in range(CHUNK):
                pltpu.make_async_copy(ut_ref.at[:, pl.ds(0, 128)],
                                      utile.at[p, k], sem_u).wait()
                pltpu.make_async_copy(it_ref.at[:, pl.ds(0, 128)],
                                      itile.at[p, k], sem_i).wait()

        def extract_chunk(c, acc):
            p = c % 3
            for k in range(CHUNK):
                cu, ci = ids_at(c, k)
                pu = jnp.full((LANES,), cu & 15, jnp.int32)
                pi = jnp.full((LANES,), ci & 15, jnp.int32)
                o16u = ((cu >> 4) & 7) * 16
                o16i = ((ci >> 4) & 7) * 16
                s = jnp.zeros((LANES,), jnp.float32)
                for d in range(EMBED_DIM):
                    uv = utile[p, k, d, pl.ds(o16u, LANES)]
                    iv = itile[p, k, d, pl.ds(o16i, LANES)]
                    s = s + shuffle(uv, pu) * iv
                acc = jnp.where(lane == (c % 4) * CHUNK + k,
                                shuffle(s, pi), acc)
            return acc

        accs = [jnp.zeros((LANES,), jnp.float32) for _ in range(2)]
        fire_chunk(0)
        fire_chunk(1)
        for c in range(8):
            if c + 2 < 8:
                fire_chunk(c + 2)
            wait_chunk(c)
            accs[c // 4] = extract_chunk(c, accs[c // 4])

        for i in range(2):
            out_v[pl.ds((g + i) * LANES, LANES)] = (
                5.0 / (1.0 + jnp.exp(-accs[i])))
        return 0

    lax.fori_loop(0, GROUPS // 2, group_body, 0)

    pltpu.sync_copy(out_v, out_ref.at[wid])


def kernel(user_ids, item_ids, user_table, item_table):
    uids = user_ids.astype(jnp.int32)
    iids = item_ids.astype(jnp.int32)
    ut = user_table.T
    it = item_table.T

    mesh = plsc.VectorSubcoreMesh(core_axis_name="c", subcore_axis_name="s")
    f = pl.kernel(
        _body,
        out_type=jax.ShapeDtypeStruct((NUM_WORKERS, ROWS_PER_WORKER),
                                      jnp.float32),
        mesh=mesh,
        scratch_types=[
            pltpu.VMEM((ROWS_PER_WORKER,), jnp.int32),
            pltpu.VMEM((ROWS_PER_WORKER,), jnp.int32),
            pltpu.VMEM((3, CHUNK, EMBED_DIM, 128), jnp.float32),
            pltpu.VMEM((3, CHUNK, EMBED_DIM, 128), jnp.float32),
            pltpu.VMEM((ROWS_PER_WORKER,), jnp.float32),
            pltpu.SemaphoreType.DMA,
            pltpu.SemaphoreType.DMA,
        ],
        compiler_params=pltpu.CompilerParams(use_tc_tiling_on_sc=True),
    )
    return f(uids, iids, ut, it).reshape(BATCH)
